# baseline (speedup 1.0000x reference)
import functools

import jax
import jax.numpy as jnp
from jax import lax
from jax.experimental import pallas as pl
from jax.experimental.pallas import tpu as pltpu
from jax.experimental.pallas import tpu_sc as plsc

_NUM_FRAMES = 100000
_NUM_CHANNELS = 64
_BATCH = 16384

_info = plsc.get_sparse_core_info()
_NC, _NS = _info.num_cores, _info.num_subcores
_NW = _NC * _NS
_BPW = _BATCH // _NW
_CHUNK = 128
_NCHUNK = _BPW // _CHUNK


@functools.partial(
    pl.kernel,
    mesh=plsc.VectorSubcoreMesh(core_axis_name="c", subcore_axis_name="s"),
    out_type=jax.ShapeDtypeStruct((_BATCH, _NUM_CHANNELS), jnp.float32),
    scratch_types=[
        pltpu.VMEM((_BPW,), jnp.int32),
        pltpu.VMEM((_BPW, _NUM_CHANNELS), jnp.float32),
        pltpu.SemaphoreType.DMA,
        pltpu.SemaphoreType.DMA,
    ],
    compiler_params=pltpu.CompilerParams(use_tc_tiling_on_sc=False),
)
def _gather_kernel(ids_hbm, data_hbm, out_hbm, idx_v, rows_v, gsem, wsem):
    wid = lax.axis_index("s") * _NC + lax.axis_index("c")
    base = wid * _BPW
    pltpu.sync_copy(ids_hbm.at[pl.ds(base, _BPW)], idx_v)
    gathers = [
        pltpu.async_copy(
            data_hbm.at[idx_v.at[pl.ds(c * _CHUNK, _CHUNK)]],
            rows_v.at[pl.ds(c * _CHUNK, _CHUNK)],
            gsem,
        )
        for c in range(_NCHUNK)
    ]
    writes = []
    for c in range(_NCHUNK):
        gathers[c].wait()
        writes.append(
            pltpu.async_copy(
                rows_v.at[pl.ds(c * _CHUNK, _CHUNK)],
                out_hbm.at[pl.ds(base + c * _CHUNK, _CHUNK)],
                wsem,
            )
        )
    for w in writes:
        w.wait()


def kernel(ids, data):
    return _gather_kernel(ids, data)


# speedup vs baseline: 1.4858x; 1.4858x over previous
import functools

import jax
import jax.numpy as jnp
from jax import lax
from jax.experimental import pallas as pl
from jax.experimental.pallas import tpu as pltpu
from jax.experimental.pallas import tpu_sc as plsc

_NUM_FRAMES = 100000
_NUM_CHANNELS = 64
_BATCH = 16384

_info = plsc.get_sparse_core_info()
_NC, _NS = _info.num_cores, _info.num_subcores
_NW = _NC * _NS
_BPW = _BATCH // _NW
_UNROLL = 16


@functools.partial(
    pl.kernel,
    mesh=plsc.VectorSubcoreMesh(core_axis_name="c", subcore_axis_name="s"),
    out_type=jax.ShapeDtypeStruct((_BATCH, _NUM_CHANNELS), jnp.float32),
    scratch_types=[
        pltpu.VMEM((_BPW,), jnp.int32),
        pltpu.VMEM((_BPW, _NUM_CHANNELS), jnp.float32),
        pltpu.SemaphoreType.DMA,
        pltpu.SemaphoreType.DMA,
    ],
)
def _gather_kernel(ids_hbm, data_hbm, out_hbm, idx_v, rows_v, gsem, wsem):
    wid = lax.axis_index("s") * _NC + lax.axis_index("c")
    base = wid * _BPW
    pltpu.sync_copy(ids_hbm.at[pl.ds(base, _BPW)], idx_v)

    def issue(step):
        vec = idx_v[pl.ds(step * _UNROLL, _UNROLL)]
        for j in range(_UNROLL):
            i = step * _UNROLL + j
            rid = vec[j]
            pltpu.async_copy(data_hbm.at[rid], rows_v.at[i], gsem)

    pl.loop(0, _BPW // _UNROLL)(issue)

    def drain(step):
        for j in range(_UNROLL):
            i = step * _UNROLL + j
            pltpu.make_async_copy(data_hbm.at[0], rows_v.at[i], gsem).wait()

    pl.loop(0, _BPW // _UNROLL)(drain)
    pltpu.async_copy(rows_v, out_hbm.at[pl.ds(base, _BPW)], wsem).wait()


def kernel(ids, data):
    return _gather_kernel(ids, data)


# speedup vs baseline: 1.4979x; 1.0081x over previous
import functools

import jax
import jax.numpy as jnp
from jax import lax
from jax.experimental import pallas as pl
from jax.experimental.pallas import tpu as pltpu
from jax.experimental.pallas import tpu_sc as plsc

_NUM_FRAMES = 100000
_NUM_CHANNELS = 64
_BATCH = 16384

_info = plsc.get_sparse_core_info()
_NC, _NS = _info.num_cores, _info.num_subcores
_NW = _NC * _NS
_BPW = _BATCH // _NW
_UNROLL = 16


@functools.partial(
    pl.kernel,
    mesh=plsc.VectorSubcoreMesh(core_axis_name="c", subcore_axis_name="s"),
    out_type=jax.ShapeDtypeStruct((_BATCH, _NUM_CHANNELS), jnp.float32),
    scratch_types=[
        pltpu.VMEM((_BPW,), jnp.int32),
        pltpu.VMEM((_BPW, _NUM_CHANNELS), jnp.float32),
        pltpu.SemaphoreType.DMA,
        pltpu.SemaphoreType.DMA,
    ],
    compiler_params=pltpu.CompilerParams(
        disable_bounds_checks=True,
        disable_semaphore_checks=True,
    ),
)
def _gather_kernel(ids_hbm, data_hbm, out_hbm, idx_v, rows_v, gsem, wsem):
    wid = lax.axis_index("s") * _NC + lax.axis_index("c")
    base = wid * _BPW
    pltpu.sync_copy(ids_hbm.at[pl.ds(base, _BPW)], idx_v)

    def issue(step):
        vec = idx_v[pl.ds(step * _UNROLL, _UNROLL)]
        for j in range(_UNROLL):
            i = step * _UNROLL + j
            rid = vec[j]
            pltpu.async_copy(data_hbm.at[rid], rows_v.at[i], gsem)

    pl.loop(0, _BPW // _UNROLL)(issue)

    def drain(step):
        for j in range(_UNROLL):
            i = step * _UNROLL + j
            pltpu.make_async_copy(data_hbm.at[0], rows_v.at[i], gsem).wait()

    pl.loop(0, _BPW // _UNROLL)(drain)
    pltpu.async_copy(rows_v, out_hbm.at[pl.ds(base, _BPW)], wsem).wait()


def kernel(ids, data):
    return _gather_kernel(ids, data)


# speedup vs baseline: 1.8499x; 1.2350x over previous
import functools

import jax
import jax.numpy as jnp
from jax import lax
from jax.experimental import pallas as pl
from jax.experimental.pallas import tpu as pltpu
from jax.experimental.pallas import tpu_sc as plsc

_NUM_FRAMES = 100000
_NUM_CHANNELS = 64
_BATCH = 16384

_info = plsc.get_sparse_core_info()
_NC, _NS, _L = _info.num_cores, _info.num_subcores, _info.num_lanes
_NW = _NC * _NS
_CPW = _NUM_CHANNELS // _NW
_BCHUNK = 4096
_NBCHUNK = _BATCH // _BCHUNK


@functools.partial(
    pl.kernel,
    mesh=plsc.VectorSubcoreMesh(core_axis_name="c", subcore_axis_name="s"),
    out_type=jax.ShapeDtypeStruct((_NUM_CHANNELS, _BATCH), jnp.float32),
    scratch_types=[
        pltpu.VMEM((_NUM_FRAMES,), jnp.float32),
        pltpu.VMEM((_BCHUNK,), jnp.int32),
        pltpu.VMEM((_BCHUNK,), jnp.float32),
    ],
    compiler_params=pltpu.CompilerParams(
        disable_bounds_checks=True,
        disable_semaphore_checks=True,
        needs_layout_passes=False,
    ),
)
def _gather_kernel(ids_hbm, data_t_hbm, out_t_hbm, row_v, idx_v, val_v):
    wid = lax.axis_index("s") * _NC + lax.axis_index("c")

    for cc in range(_CPW):
        ch = wid * _CPW + cc
        pltpu.sync_copy(data_t_hbm.at[ch], row_v)
        for b in range(_NBCHUNK):
            pltpu.sync_copy(ids_hbm.at[pl.ds(b * _BCHUNK, _BCHUNK)], idx_v)

            def gather_block(k):
                idx = idx_v[pl.ds(k * _L, _L)]
                val_v[pl.ds(k * _L, _L)] = plsc.load_gather(row_v, [idx])

            pl.loop(0, _BCHUNK // _L)(gather_block)
            pltpu.sync_copy(val_v, out_t_hbm.at[ch, pl.ds(b * _BCHUNK, _BCHUNK)])


def kernel(ids, data):
    out_t = _gather_kernel(ids, data.T)
    return out_t.T
